# Initial kernel scaffold; baseline (speedup 1.0000x reference)
import functools

import jax
import jax.numpy as jnp
from jax import lax
from jax.experimental import pallas as pl
from jax.experimental.pallas import tpu as pltpu
from jax.experimental.pallas import tpu_sc as plsc

VOCAB = 1000000
D = 64
B = 16384
NEG = 5

NC = 2
NS = 16
NW = NC * NS
L = 16
BPW = B // NW
CH = 128
NR = BPW // CH
GPR = CH // L


def _sc_body(u3, v3, n4, U_hbm, V_hbm, score_out, neg_out,
             uidx, vidx, nidx, urows, vrows, nrows, sbuf, nbuf, sem):
    wid = lax.axis_index("s") * NC + lax.axis_index("c")
    base = wid * BPW

    pltpu.sync_copy(u3.at[wid], uidx)
    pltpu.sync_copy(v3.at[wid], vidx)
    pltpu.sync_copy(n4.at[wid], nidx)

    lane = lax.iota(jnp.int32, L)

    for r in range(NR):
        cps = [
            pltpu.async_copy(U_hbm.at[uidx.at[r]], urows, sem),
            pltpu.async_copy(V_hbm.at[vidx.at[r]], vrows, sem),
        ]
        for k in range(NEG):
            cps.append(pltpu.async_copy(V_hbm.at[nidx.at[k, r]], nrows.at[k], sem))
        for c in cps:
            c.wait()

        def group_body(go, _):
            rows = go * L + lane

            def d_body(d, carry):
                acc_p, acc_n = carry
                dcol = jnp.full((L,), d, jnp.int32)
                ucol = plsc.load_gather(urows, [rows, dcol])
                vcol = plsc.load_gather(vrows, [rows, dcol])
                ncol = plsc.load_gather(nrows, [jnp.zeros((L,), jnp.int32), rows, dcol])
                for k in range(1, NEG):
                    ncol = ncol + plsc.load_gather(
                        nrows, [jnp.full((L,), k, jnp.int32), rows, dcol])
                return acc_p + ucol * vcol, acc_n + ucol * ncol

            z = jnp.zeros((L,), jnp.float32)
            acc_p, acc_n = lax.fori_loop(0, D, d_body, (z, z))
            off = r * CH + go * L
            sbuf[pl.ds(off, L)] = acc_p
            nbuf[pl.ds(off, L)] = acc_n
            return 0

        lax.fori_loop(0, GPR, group_body, 0)

    pltpu.sync_copy(sbuf, score_out.at[pl.ds(base, BPW)])
    pltpu.sync_copy(nbuf, neg_out.at[pl.ds(base, BPW)])


def _loss_body(s_ref, n_ref, o_ref):
    s = s_ref[...]
    n = -n_ref[...]

    def ls(x):
        return jnp.minimum(x, 0.0) - jnp.log1p(jnp.exp(-jnp.abs(x)))

    o_ref[0, 0] = -(jnp.sum(ls(s) + ls(n))) / B


def kernel(u, v, neg_v, U, V):
    u3 = u.astype(jnp.int32).reshape(NW, NR, CH)
    v3 = v.astype(jnp.int32).reshape(NW, NR, CH)
    n4 = (neg_v.astype(jnp.int32).T.reshape(NEG, NW, NR, CH)
          .transpose(1, 0, 2, 3))

    mesh = plsc.VectorSubcoreMesh(core_axis_name="c", subcore_axis_name="s")
    score, negsc = pl.kernel(
        _sc_body,
        out_type=[
            jax.ShapeDtypeStruct((B,), jnp.float32),
            jax.ShapeDtypeStruct((B,), jnp.float32),
        ],
        mesh=mesh,
        scratch_types=[
            pltpu.VMEM((NR, CH), jnp.int32),
            pltpu.VMEM((NR, CH), jnp.int32),
            pltpu.VMEM((NEG, NR, CH), jnp.int32),
            pltpu.VMEM((CH, D), jnp.float32),
            pltpu.VMEM((CH, D), jnp.float32),
            pltpu.VMEM((NEG, CH, D), jnp.float32),
            pltpu.VMEM((BPW,), jnp.float32),
            pltpu.VMEM((BPW,), jnp.float32),
            pltpu.SemaphoreType.DMA,
        ],
    )(u3, v3, n4, U, V)

    loss = pl.pallas_call(
        _loss_body,
        out_shape=jax.ShapeDtypeStruct((1, 1), jnp.float32),
    )(score.reshape(B // CH, CH), negsc.reshape(B // CH, CH))
    return loss[0, 0]


# speedup vs baseline: 1.5956x; 1.5956x over previous
import functools

import jax
import jax.numpy as jnp
from jax import lax
from jax.experimental import pallas as pl
from jax.experimental.pallas import tpu as pltpu
from jax.experimental.pallas import tpu_sc as plsc

VOCAB = 1000000
D = 64
B = 16384
NEG = 5

NC = 2
NS = 16
NW = NC * NS
L = 16
BPW = B // NW
CH = 128
NR = BPW // CH
GPR = CH // L


def _sc_body(u3, v3, n4, U_hbm, V_hbm, score_out, neg_out,
             uidx, vidx, nidx, urows, vrows, nrows, sbuf, nbuf, sem):
    wid = lax.axis_index("s") * NC + lax.axis_index("c")
    base = wid * BPW

    pltpu.sync_copy(u3.at[wid], uidx)
    pltpu.sync_copy(v3.at[wid], vidx)
    pltpu.sync_copy(n4.at[wid], nidx)

    lane = lax.iota(jnp.int32, L)

    for r in range(NR):
        cps = [
            pltpu.async_copy(U_hbm.at[uidx.at[r]], urows, sem),
            pltpu.async_copy(V_hbm.at[vidx.at[r]], vrows, sem),
        ]
        for k in range(NEG):
            cps.append(pltpu.async_copy(V_hbm.at[nidx.at[k, r]], nrows.at[k], sem))
        for c in cps:
            c.wait()

        def group_body(go, _):
            rows = go * L + lane

            def d_body(d, carry):
                acc_p, acc_n = carry
                dcol = jnp.full((L,), d, jnp.int32)
                ucol = plsc.load_gather(urows, [rows, dcol])
                vcol = plsc.load_gather(vrows, [rows, dcol])
                ncol = plsc.load_gather(nrows, [jnp.zeros((L,), jnp.int32), rows, dcol])
                for k in range(1, NEG):
                    ncol = ncol + plsc.load_gather(
                        nrows, [jnp.full((L,), k, jnp.int32), rows, dcol])
                return acc_p + ucol * vcol, acc_n + ucol * ncol

            z = jnp.zeros((L,), jnp.float32)
            acc_p, acc_n = lax.fori_loop(0, D, d_body, (z, z))
            off = r * CH + go * L
            sbuf[pl.ds(off, L)] = acc_p
            nbuf[pl.ds(off, L)] = acc_n
            return 0

        lax.fori_loop(0, GPR, group_body, 0)

    pltpu.sync_copy(sbuf, score_out.at[pl.ds(base, BPW)])
    pltpu.sync_copy(nbuf, neg_out.at[pl.ds(base, BPW)])


def _loss_body(s_ref, n_ref, o_ref):
    s = s_ref[...]
    n = -n_ref[...]

    def ls(x):
        return jnp.minimum(x, 0.0) - jnp.log1p(jnp.exp(-jnp.abs(x)))

    o_ref[...] = (-(jnp.sum(ls(s) + ls(n))) / B).reshape(1, 1)


def kernel(u, v, neg_v, U, V):
    u3 = u.astype(jnp.int32).reshape(NW, NR, CH)
    v3 = v.astype(jnp.int32).reshape(NW, NR, CH)
    n4 = (neg_v.astype(jnp.int32).T.reshape(NEG, NW, NR, CH)
          .transpose(1, 0, 2, 3))

    mesh = plsc.VectorSubcoreMesh(core_axis_name="c", subcore_axis_name="s")
    score, negsc = pl.kernel(
        _sc_body,
        out_type=[
            jax.ShapeDtypeStruct((B,), jnp.float32),
            jax.ShapeDtypeStruct((B,), jnp.float32),
        ],
        mesh=mesh,
        compiler_params=pltpu.CompilerParams(
            needs_layout_passes=False, use_tc_tiling_on_sc=False),
        scratch_types=[
            pltpu.VMEM((NR, CH), jnp.int32),
            pltpu.VMEM((NR, CH), jnp.int32),
            pltpu.VMEM((NEG, NR, CH), jnp.int32),
            pltpu.VMEM((CH, D), jnp.float32),
            pltpu.VMEM((CH, D), jnp.float32),
            pltpu.VMEM((NEG, CH, D), jnp.float32),
            pltpu.VMEM((BPW,), jnp.float32),
            pltpu.VMEM((BPW,), jnp.float32),
            pltpu.SemaphoreType.DMA,
        ],
    )(u3, v3, n4, U, V)

    loss = pl.pallas_call(
        _loss_body,
        out_shape=jax.ShapeDtypeStruct((1, 1), jnp.float32),
    )(score.reshape(B // CH, CH), negsc.reshape(B // CH, CH))
    return loss[0, 0]


# speedup vs baseline: 2.4328x; 1.5247x over previous
import jax
import jax.numpy as jnp
from jax import lax
from jax.experimental import pallas as pl
from jax.experimental.pallas import tpu as pltpu
from jax.experimental.pallas import tpu_sc as plsc

VOCAB = 1000000
D = 64
B = 16384
NEG = 5

NC = 2
NS = 16
NW = NC * NS
L = 16
BPW = B // NW
CH = 128
NR = BPW // CH
GPR = CH // L

TBLK = 2048


def _wbuild_body(ut_ref, vt_ref, o_ref):
    o_ref[:, 0:D] = jnp.transpose(ut_ref[...])
    o_ref[:, D:2 * D] = jnp.transpose(vt_ref[...])


def _sc_body(uv3, n3, W_hbm, out_hbm,
             uvidx, nidx, rows_v, sbuf, sem):
    wid = lax.axis_index("s") * NC + lax.axis_index("c")

    pltpu.sync_copy(uv3.at[wid], uvidx)
    pltpu.sync_copy(n3.at[wid], nidx)

    lane = lax.iota(jnp.int32, L)

    for r in range(NR):
        cps = [
            pltpu.async_copy(W_hbm.at[uvidx.at[r]], rows_v.at[0], sem),
            pltpu.async_copy(W_hbm.at[uvidx.at[NR + r]], rows_v.at[1], sem),
        ]
        for k in range(NEG):
            cps.append(pltpu.async_copy(
                W_hbm.at[nidx.at[k * NR + r]], rows_v.at[2 + k], sem))
        for c in cps:
            c.wait()

        def group_body(go, _):
            rows = go * L + lane

            def d_body(d, carry):
                acc_p, acc_n = carry
                du = jnp.full((L,), d, jnp.int32)
                dv = jnp.full((L,), D + d, jnp.int32)
                zero = jnp.zeros((L,), jnp.int32)
                ucol = plsc.load_gather(rows_v, [zero, rows, du])
                vcol = plsc.load_gather(rows_v, [jnp.full((L,), 1, jnp.int32), rows, dv])
                ncol = plsc.load_gather(rows_v, [jnp.full((L,), 2, jnp.int32), rows, dv])
                for k in range(1, NEG):
                    ncol = ncol + plsc.load_gather(
                        rows_v, [jnp.full((L,), 2 + k, jnp.int32), rows, dv])
                return acc_p + ucol * vcol, acc_n + ucol * ncol

            z = jnp.zeros((L,), jnp.float32)
            acc_p, acc_n = lax.fori_loop(0, D, d_body, (z, z))
            sbuf[r, pl.ds(go * L, L)] = acc_p
            sbuf[NR + r, pl.ds(go * L, L)] = acc_n
            return 0

        lax.fori_loop(0, GPR, group_body, 0)

    pltpu.sync_copy(sbuf, out_hbm.at[wid])


def _loss_body(x_ref, o_ref):
    s = x_ref[:, 0:NR, :]
    n = -x_ref[:, NR:2 * NR, :]

    def ls(x):
        return jnp.minimum(x, 0.0) - jnp.log1p(jnp.exp(-jnp.abs(x)))

    o_ref[...] = (-(jnp.sum(ls(s) + ls(n))) / B).reshape(1, 1)


def kernel(u, v, neg_v, U, V):
    W = pl.pallas_call(
        _wbuild_body,
        grid=(pl.cdiv(VOCAB, TBLK),),
        in_specs=[
            pl.BlockSpec((D, TBLK), lambda j: (0, j)),
            pl.BlockSpec((D, TBLK), lambda j: (0, j)),
        ],
        out_specs=pl.BlockSpec((TBLK, 2 * D), lambda j: (j, 0)),
        out_shape=jax.ShapeDtypeStruct((VOCAB, 2 * D), jnp.float32),
    )(U.T, V.T)

    uv3 = jnp.concatenate(
        [u.astype(jnp.int32).reshape(NW, NR, CH),
         v.astype(jnp.int32).reshape(NW, NR, CH)], axis=1)
    n3 = (neg_v.astype(jnp.int32).T.reshape(NEG, NW, NR, CH)
          .transpose(1, 0, 2, 3).reshape(NW, NEG * NR, CH))

    mesh = plsc.VectorSubcoreMesh(core_axis_name="c", subcore_axis_name="s")
    packed = pl.kernel(
        _sc_body,
        out_type=jax.ShapeDtypeStruct((NW, 2 * NR, CH), jnp.float32),
        mesh=mesh,
        compiler_params=pltpu.CompilerParams(needs_layout_passes=False),
        scratch_types=[
            pltpu.VMEM((2 * NR, CH), jnp.int32),
            pltpu.VMEM((NEG * NR, CH), jnp.int32),
            pltpu.VMEM((2 + NEG, CH, 2 * D), jnp.float32),
            pltpu.VMEM((2 * NR, CH), jnp.float32),
            pltpu.SemaphoreType.DMA,
        ],
    )(uv3, n3, W)

    loss = pl.pallas_call(
        _loss_body,
        out_shape=jax.ShapeDtypeStruct((1, 1), jnp.float32),
    )(packed)
    return loss[0, 0]


# speedup vs baseline: 2.5375x; 1.0431x over previous
import jax
import jax.numpy as jnp
from jax import lax
from jax.experimental import pallas as pl
from jax.experimental.pallas import tpu as pltpu
from jax.experimental.pallas import tpu_sc as plsc

VOCAB = 1000000
D = 64
B = 16384
NEG = 5

NC = 2
NS = 16
NW = NC * NS
L = 16
BPW = B // NW
CH = 128
NR = BPW // CH
GPR = CH // L

TBLK = 2048


def _wbuild_body(ut_ref, vt_ref, e1_ref, e2_ref, o_ref):
    dn = (((0,), (0,)), ((), ()))
    o_ref[...] = (
        lax.dot_general(ut_ref[...], e1_ref[...], dn,
                        preferred_element_type=jnp.float32)
        + lax.dot_general(vt_ref[...], e2_ref[...], dn,
                          preferred_element_type=jnp.float32))


def _sc_body(uv3, n3, W_hbm, out_hbm,
             uvidx, nidx, rows_v, sbuf, sem):
    wid = lax.axis_index("s") * NC + lax.axis_index("c")

    pltpu.sync_copy(uv3.at[wid], uvidx)
    pltpu.sync_copy(n3.at[wid], nidx)

    lane = lax.iota(jnp.int32, L)

    for r in range(NR):
        cps = [
            pltpu.async_copy(W_hbm.at[uvidx.at[r]], rows_v.at[0], sem),
            pltpu.async_copy(W_hbm.at[uvidx.at[NR + r]], rows_v.at[1], sem),
        ]
        for k in range(NEG):
            cps.append(pltpu.async_copy(
                W_hbm.at[nidx.at[k * NR + r]], rows_v.at[2 + k], sem))
        for c in cps:
            c.wait()

        def group_body(go, _):
            rows = go * L + lane

            def d_body(d, carry):
                acc_p, acc_n = carry
                du = jnp.full((L,), d, jnp.int32)
                dv = jnp.full((L,), D + d, jnp.int32)
                zero = jnp.zeros((L,), jnp.int32)
                ucol = plsc.load_gather(rows_v, [zero, rows, du])
                vcol = plsc.load_gather(rows_v, [jnp.full((L,), 1, jnp.int32), rows, dv])
                ncol = plsc.load_gather(rows_v, [jnp.full((L,), 2, jnp.int32), rows, dv])
                for k in range(1, NEG):
                    ncol = ncol + plsc.load_gather(
                        rows_v, [jnp.full((L,), 2 + k, jnp.int32), rows, dv])
                return acc_p + ucol * vcol, acc_n + ucol * ncol

            z = jnp.zeros((L,), jnp.float32)
            acc_p, acc_n = lax.fori_loop(0, D, d_body, (z, z))
            sbuf[r, pl.ds(go * L, L)] = acc_p
            sbuf[NR + r, pl.ds(go * L, L)] = acc_n
            return 0

        lax.fori_loop(0, GPR, group_body, 0)

    pltpu.sync_copy(sbuf, out_hbm.at[wid])


def _loss_body(x_ref, o_ref):
    s = x_ref[:, 0:NR, :]
    n = -x_ref[:, NR:2 * NR, :]

    def ls(x):
        return jnp.minimum(x, 0.0) - jnp.log1p(jnp.exp(-jnp.abs(x)))

    o_ref[...] = (-(jnp.sum(ls(s) + ls(n))) / B).reshape(1, 1)


def kernel(u, v, neg_v, U, V):
    eye = jnp.eye(D, dtype=jnp.float32)
    zer = jnp.zeros((D, D), jnp.float32)
    e1 = jnp.concatenate([eye, zer], axis=1)
    e2 = jnp.concatenate([zer, eye], axis=1)
    W = pl.pallas_call(
        _wbuild_body,
        grid=(pl.cdiv(VOCAB, TBLK),),
        in_specs=[
            pl.BlockSpec((D, TBLK), lambda j: (0, j)),
            pl.BlockSpec((D, TBLK), lambda j: (0, j)),
            pl.BlockSpec((D, 2 * D), lambda j: (0, 0)),
            pl.BlockSpec((D, 2 * D), lambda j: (0, 0)),
        ],
        out_specs=pl.BlockSpec((TBLK, 2 * D), lambda j: (j, 0)),
        out_shape=jax.ShapeDtypeStruct((VOCAB, 2 * D), jnp.float32),
    )(U.T, V.T, e1, e2)

    uv3 = jnp.concatenate(
        [u.astype(jnp.int32).reshape(NW, NR, CH),
         v.astype(jnp.int32).reshape(NW, NR, CH)], axis=1)
    n3 = (neg_v.astype(jnp.int32).T.reshape(NEG, NW, NR, CH)
          .transpose(1, 0, 2, 3).reshape(NW, NEG * NR, CH))

    mesh = plsc.VectorSubcoreMesh(core_axis_name="c", subcore_axis_name="s")
    packed = pl.kernel(
        _sc_body,
        out_type=jax.ShapeDtypeStruct((NW, 2 * NR, CH), jnp.float32),
        mesh=mesh,
        compiler_params=pltpu.CompilerParams(needs_layout_passes=False),
        scratch_types=[
            pltpu.VMEM((2 * NR, CH), jnp.int32),
            pltpu.VMEM((NEG * NR, CH), jnp.int32),
            pltpu.VMEM((2 + NEG, CH, 2 * D), jnp.float32),
            pltpu.VMEM((2 * NR, CH), jnp.float32),
            pltpu.SemaphoreType.DMA,
        ],
    )(uv3, n3, W)

    loss = pl.pallas_call(
        _loss_body,
        out_shape=jax.ShapeDtypeStruct((1, 1), jnp.float32),
    )(packed)
    return loss[0, 0]


# speedup vs baseline: 3.5712x; 1.4074x over previous
import jax
import jax.numpy as jnp
from jax import lax
from jax.experimental import pallas as pl
from jax.experimental.pallas import tpu as pltpu
from jax.experimental.pallas import tpu_sc as plsc

VOCAB = 1000000
D = 64
B = 16384
NEG = 5

NC = 2
NS = 16
NW = NC * NS
L = 16
BPW = B // NW
CH = 128
NR = BPW // CH
GPR = CH // L

TBLK = 8192


def _wbuild_body(ut_ref, vt_ref, e1_ref, e2_ref, o_ref):
    dn = (((0,), (0,)), ((), ()))
    o_ref[...] = (
        lax.dot_general(ut_ref[...], e1_ref[...], dn,
                        preferred_element_type=jnp.float32)
        + lax.dot_general(vt_ref[...], e2_ref[...], dn,
                          preferred_element_type=jnp.float32))


def _sc_body(uv3, n3, W_hbm, out_hbm,
             uvidx, nidx, rows_v, sbuf, sem):
    wid = lax.axis_index("s") * NC + lax.axis_index("c")

    pltpu.sync_copy(uv3.at[wid], uvidx)
    pltpu.sync_copy(n3.at[wid], nidx)

    lane = lax.iota(jnp.int32, L)

    for r in range(NR):
        cps = [
            pltpu.async_copy(W_hbm.at[uvidx.at[r]], rows_v.at[0], sem),
            pltpu.async_copy(W_hbm.at[uvidx.at[NR + r]], rows_v.at[1], sem),
        ]
        for k in range(NEG):
            cps.append(pltpu.async_copy(
                W_hbm.at[nidx.at[k * NR + r]], rows_v.at[2 + k], sem))
        for c in cps:
            c.wait()

        def group_body(go, _):
            rows = go * L + lane

            def d_body(d, carry):
                acc_p, acc_n = carry
                du = jnp.full((L,), d, jnp.int32)
                dv = jnp.full((L,), D + d, jnp.int32)
                zero = jnp.zeros((L,), jnp.int32)
                ucol = plsc.load_gather(rows_v, [zero, rows, du])
                vcol = plsc.load_gather(rows_v, [jnp.full((L,), 1, jnp.int32), rows, dv])
                ncol = plsc.load_gather(rows_v, [jnp.full((L,), 2, jnp.int32), rows, dv])
                for k in range(1, NEG):
                    ncol = ncol + plsc.load_gather(
                        rows_v, [jnp.full((L,), 2 + k, jnp.int32), rows, dv])
                return acc_p + ucol * vcol, acc_n + ucol * ncol

            z = jnp.zeros((L,), jnp.float32)
            acc_p, acc_n = lax.fori_loop(0, D, d_body, (z, z))
            sbuf[r, pl.ds(go * L, L)] = acc_p
            sbuf[NR + r, pl.ds(go * L, L)] = acc_n
            return 0

        lax.fori_loop(0, GPR, group_body, 0)

    pltpu.sync_copy(sbuf, out_hbm.at[wid])


def _loss_body(x_ref, o_ref):
    s = x_ref[:, 0:NR, :]
    n = -x_ref[:, NR:2 * NR, :]

    def ls(x):
        return jnp.minimum(x, 0.0) - jnp.log1p(jnp.exp(-jnp.abs(x)))

    o_ref[...] = (-(jnp.sum(ls(s) + ls(n))) / B).reshape(1, 1)


def kernel(u, v, neg_v, U, V):
    eye = jnp.eye(D, dtype=jnp.float32)
    zer = jnp.zeros((D, D), jnp.float32)
    e1 = jnp.concatenate([eye, zer], axis=1)
    e2 = jnp.concatenate([zer, eye], axis=1)
    W = pl.pallas_call(
        _wbuild_body,
        grid=(pl.cdiv(VOCAB, TBLK),),
        in_specs=[
            pl.BlockSpec((D, TBLK), lambda j: (0, j)),
            pl.BlockSpec((D, TBLK), lambda j: (0, j)),
            pl.BlockSpec((D, 2 * D), lambda j: (0, 0)),
            pl.BlockSpec((D, 2 * D), lambda j: (0, 0)),
        ],
        out_specs=pl.BlockSpec((TBLK, 2 * D), lambda j: (j, 0)),
        out_shape=jax.ShapeDtypeStruct((VOCAB, 2 * D), jnp.float32),
    )(U.T, V.T, e1, e2)

    uv3 = jnp.concatenate(
        [u.astype(jnp.int32).reshape(NW, NR, CH),
         v.astype(jnp.int32).reshape(NW, NR, CH)], axis=1)
    n3 = (neg_v.astype(jnp.int32).T.reshape(NEG, NW, NR, CH)
          .transpose(1, 0, 2, 3).reshape(NW, NEG * NR, CH))

    mesh = plsc.VectorSubcoreMesh(core_axis_name="c", subcore_axis_name="s")
    packed = pl.kernel(
        _sc_body,
        out_type=jax.ShapeDtypeStruct((NW, 2 * NR, CH), jnp.float32),
        mesh=mesh,
        compiler_params=pltpu.CompilerParams(needs_layout_passes=False),
        scratch_types=[
            pltpu.VMEM((2 * NR, CH), jnp.int32),
            pltpu.VMEM((NEG * NR, CH), jnp.int32),
            pltpu.VMEM((2 + NEG, CH, 2 * D), jnp.float32),
            pltpu.VMEM((2 * NR, CH), jnp.float32),
            pltpu.SemaphoreType.DMA,
        ],
    )(uv3, n3, W)

    loss = pl.pallas_call(
        _loss_body,
        out_shape=jax.ShapeDtypeStruct((1, 1), jnp.float32),
    )(packed)
    return loss[0, 0]


# speedup vs baseline: 3.8595x; 1.0807x over previous
import jax
import jax.numpy as jnp
from jax import lax
from jax.experimental import pallas as pl
from jax.experimental.pallas import tpu as pltpu
from jax.experimental.pallas import tpu_sc as plsc

VOCAB = 1000000
D = 64
B = 16384
NEG = 5

NC = 2
NS = 16
NW = NC * NS
L = 16
BPW = B // NW
CH = 128
NR = BPW // CH
GPR = CH // L

TBLK = 16384


def _wbuild_body(ut_ref, vt_ref, e1_ref, e2_ref, o_ref):
    dn = (((0,), (0,)), ((), ()))
    o_ref[...] = (
        lax.dot_general(ut_ref[...], e1_ref[...], dn,
                        preferred_element_type=jnp.float32)
        + lax.dot_general(vt_ref[...], e2_ref[...], dn,
                          preferred_element_type=jnp.float32))


def _sc_body(uv3, n3, W_hbm, out_hbm,
             uvidx, nidx, rows_v, sbuf, sem):
    wid = lax.axis_index("s") * NC + lax.axis_index("c")

    pltpu.sync_copy(uv3.at[wid], uvidx)
    pltpu.sync_copy(n3.at[wid], nidx)

    lane = lax.iota(jnp.int32, L)

    for r in range(NR):
        cps = [
            pltpu.async_copy(W_hbm.at[uvidx.at[r]], rows_v.at[0], sem),
            pltpu.async_copy(W_hbm.at[uvidx.at[NR + r]], rows_v.at[1], sem),
        ]
        for k in range(NEG):
            cps.append(pltpu.async_copy(
                W_hbm.at[nidx.at[k * NR + r]], rows_v.at[2 + k], sem))
        for c in cps:
            c.wait()

        def group_body(go, _):
            rows = go * L + lane

            def d_body(d, carry):
                acc_p, acc_n = carry
                du = jnp.full((L,), d, jnp.int32)
                dv = jnp.full((L,), D + d, jnp.int32)
                zero = jnp.zeros((L,), jnp.int32)
                ucol = plsc.load_gather(rows_v, [zero, rows, du])
                vcol = plsc.load_gather(rows_v, [jnp.full((L,), 1, jnp.int32), rows, dv])
                ncol = plsc.load_gather(rows_v, [jnp.full((L,), 2, jnp.int32), rows, dv])
                for k in range(1, NEG):
                    ncol = ncol + plsc.load_gather(
                        rows_v, [jnp.full((L,), 2 + k, jnp.int32), rows, dv])
                return acc_p + ucol * vcol, acc_n + ucol * ncol

            z = jnp.zeros((L,), jnp.float32)
            acc_p, acc_n = lax.fori_loop(0, D, d_body, (z, z))
            sbuf[r, pl.ds(go * L, L)] = acc_p
            sbuf[NR + r, pl.ds(go * L, L)] = acc_n
            return 0

        lax.fori_loop(0, GPR, group_body, 0)

    pltpu.sync_copy(sbuf, out_hbm.at[wid])


def _loss_body(x_ref, o_ref):
    s = x_ref[:, 0:NR, :]
    n = -x_ref[:, NR:2 * NR, :]

    def ls(x):
        return jnp.minimum(x, 0.0) - jnp.log1p(jnp.exp(-jnp.abs(x)))

    o_ref[...] = (-(jnp.sum(ls(s) + ls(n))) / B).reshape(1, 1)


def kernel(u, v, neg_v, U, V):
    eye = jnp.eye(D, dtype=jnp.float32)
    zer = jnp.zeros((D, D), jnp.float32)
    e1 = jnp.concatenate([eye, zer], axis=1)
    e2 = jnp.concatenate([zer, eye], axis=1)
    W = pl.pallas_call(
        _wbuild_body,
        grid=(pl.cdiv(VOCAB, TBLK),),
        in_specs=[
            pl.BlockSpec((D, TBLK), lambda j: (0, j)),
            pl.BlockSpec((D, TBLK), lambda j: (0, j)),
            pl.BlockSpec((D, 2 * D), lambda j: (0, 0)),
            pl.BlockSpec((D, 2 * D), lambda j: (0, 0)),
        ],
        out_specs=pl.BlockSpec((TBLK, 2 * D), lambda j: (j, 0)),
        out_shape=jax.ShapeDtypeStruct((VOCAB, 2 * D), jnp.float32),
    )(U.T, V.T, e1, e2)

    uv3 = jnp.concatenate(
        [u.astype(jnp.int32).reshape(NW, NR, CH),
         v.astype(jnp.int32).reshape(NW, NR, CH)], axis=1)
    n3 = (neg_v.astype(jnp.int32).T.reshape(NEG, NW, NR, CH)
          .transpose(1, 0, 2, 3).reshape(NW, NEG * NR, CH))

    mesh = plsc.VectorSubcoreMesh(core_axis_name="c", subcore_axis_name="s")
    packed = pl.kernel(
        _sc_body,
        out_type=jax.ShapeDtypeStruct((NW, 2 * NR, CH), jnp.float32),
        mesh=mesh,
        compiler_params=pltpu.CompilerParams(needs_layout_passes=False),
        scratch_types=[
            pltpu.VMEM((2 * NR, CH), jnp.int32),
            pltpu.VMEM((NEG * NR, CH), jnp.int32),
            pltpu.VMEM((2 + NEG, CH, 2 * D), jnp.float32),
            pltpu.VMEM((2 * NR, CH), jnp.float32),
            pltpu.SemaphoreType.DMA,
        ],
    )(uv3, n3, W)

    loss = pl.pallas_call(
        _loss_body,
        out_shape=jax.ShapeDtypeStruct((1, 1), jnp.float32),
    )(packed)
    return loss[0, 0]


# speedup vs baseline: 4.1764x; 1.0821x over previous
import jax
import jax.numpy as jnp
from jax import lax
from jax.experimental import pallas as pl
from jax.experimental.pallas import tpu as pltpu
from jax.experimental.pallas import tpu_sc as plsc

VOCAB = 1000000
D = 64
B = 16384
NEG = 5

NC = 2
NS = 16
NW = NC * NS
L = 16
BPW = B // NW
CH = 64
NR = BPW // CH
GPR = CH // L
NIC = BPW // 128

TBLK = 16384


def _wbuild_body(ut_ref, vt_ref, e1_ref, e2_ref, o_ref):
    dn = (((0,), (0,)), ((), ()))
    o_ref[...] = (
        lax.dot_general(ut_ref[...], e1_ref[...], dn,
                        preferred_element_type=jnp.float32)
        + lax.dot_general(vt_ref[...], e2_ref[...], dn,
                          preferred_element_type=jnp.float32))


def _sc_body(uv3, n3, W_hbm, out_hbm,
             uvidx, nidx, rows_v, sbuf, sem0, sem1):
    wid = lax.axis_index("s") * NC + lax.axis_index("c")

    pltpu.sync_copy(uv3.at[wid], uvidx)
    pltpu.sync_copy(n3.at[wid], nidx)

    lane = lax.iota(jnp.int32, L)
    sems = (sem0, sem1)

    def fire(r):
        b = r % 2
        s = sems[b]
        uslice = uvidx.at[r >> 1, pl.ds((r & 1) * CH, CH)]
        vslice = uvidx.at[NIC + (r >> 1), pl.ds((r & 1) * CH, CH)]
        cps = [
            pltpu.async_copy(W_hbm.at[uslice], rows_v.at[b, 0], s),
            pltpu.async_copy(W_hbm.at[vslice], rows_v.at[b, 1], s),
        ]
        for k in range(NEG):
            nslice = nidx.at[k * NIC + (r >> 1), pl.ds((r & 1) * CH, CH)]
            cps.append(pltpu.async_copy(W_hbm.at[nslice], rows_v.at[b, 2 + k], s))
        return cps

    pend = fire(0)
    for r in range(NR):
        nxt = fire(r + 1) if r + 1 < NR else []
        for c in pend:
            c.wait()
        pend = nxt
        b = r % 2
        bvec = jnp.full((L,), b, jnp.int32)
        roles = [jnp.full((L,), j, jnp.int32) for j in range(2 + NEG)]

        def group_body(go, _):
            rows = go * L + lane

            def d_body(d, carry):
                acc_p, acc_n = carry
                du = jnp.full((L,), d, jnp.int32)
                dv = du + D
                ucol = plsc.load_gather(rows_v, [bvec, roles[0], rows, du])
                vcol = plsc.load_gather(rows_v, [bvec, roles[1], rows, dv])
                ncol = plsc.load_gather(rows_v, [bvec, roles[2], rows, dv])
                for k in range(1, NEG):
                    ncol = ncol + plsc.load_gather(
                        rows_v, [bvec, roles[2 + k], rows, dv])
                return acc_p + ucol * vcol, acc_n + ucol * ncol

            z = jnp.zeros((L,), jnp.float32)
            acc_p, acc_n = lax.fori_loop(0, D, d_body, (z, z), unroll=8)
            off = (r & 1) * CH + go * L
            sbuf[r >> 1, pl.ds(off, L)] = acc_p
            sbuf[NIC + (r >> 1), pl.ds(off, L)] = acc_n
            return 0

        lax.fori_loop(0, GPR, group_body, 0)

    pltpu.sync_copy(sbuf, out_hbm.at[wid])


def _loss_body(x_ref, o_ref):
    s = x_ref[:, 0:NIC, :]
    n = -x_ref[:, NIC:2 * NIC, :]

    def ls(x):
        return jnp.minimum(x, 0.0) - jnp.log1p(jnp.exp(-jnp.abs(x)))

    o_ref[...] = (-(jnp.sum(ls(s) + ls(n))) / B).reshape(1, 1)


def kernel(u, v, neg_v, U, V):
    eye = jnp.eye(D, dtype=jnp.float32)
    zer = jnp.zeros((D, D), jnp.float32)
    e1 = jnp.concatenate([eye, zer], axis=1)
    e2 = jnp.concatenate([zer, eye], axis=1)
    W = pl.pallas_call(
        _wbuild_body,
        grid=(pl.cdiv(VOCAB, TBLK),),
        in_specs=[
            pl.BlockSpec((D, TBLK), lambda j: (0, j)),
            pl.BlockSpec((D, TBLK), lambda j: (0, j)),
            pl.BlockSpec((D, 2 * D), lambda j: (0, 0)),
            pl.BlockSpec((D, 2 * D), lambda j: (0, 0)),
        ],
        out_specs=pl.BlockSpec((TBLK, 2 * D), lambda j: (j, 0)),
        out_shape=jax.ShapeDtypeStruct((VOCAB, 2 * D), jnp.float32),
    )(U.T, V.T, e1, e2)

    uv3 = jnp.concatenate(
        [u.astype(jnp.int32).reshape(NW, NIC, 128),
         v.astype(jnp.int32).reshape(NW, NIC, 128)], axis=1)
    n3 = (neg_v.astype(jnp.int32).T.reshape(NEG, NW, NIC, 128)
          .transpose(1, 0, 2, 3).reshape(NW, NEG * NIC, 128))

    mesh = plsc.VectorSubcoreMesh(core_axis_name="c", subcore_axis_name="s")
    packed = pl.kernel(
        _sc_body,
        out_type=jax.ShapeDtypeStruct((NW, 2 * NIC, 128), jnp.float32),
        mesh=mesh,
        compiler_params=pltpu.CompilerParams(needs_layout_passes=False),
        scratch_types=[
            pltpu.VMEM((2 * NIC, 128), jnp.int32),
            pltpu.VMEM((NEG * NIC, 128), jnp.int32),
            pltpu.VMEM((2, 2 + NEG, CH, 2 * D), jnp.float32),
            pltpu.VMEM((2 * NIC, 128), jnp.float32),
            pltpu.SemaphoreType.DMA,
            pltpu.SemaphoreType.DMA,
        ],
    )(uv3, n3, W)

    loss = pl.pallas_call(
        _loss_body,
        out_shape=jax.ShapeDtypeStruct((1, 1), jnp.float32),
    )(packed)
    return loss[0, 0]


# speedup vs baseline: 4.1767x; 1.0001x over previous
import jax
import jax.numpy as jnp
from jax import lax
from jax.experimental import pallas as pl
from jax.experimental.pallas import tpu as pltpu
from jax.experimental.pallas import tpu_sc as plsc

VOCAB = 1000000
D = 64
B = 16384
NEG = 5

NC = 2
NS = 16
NW = NC * NS
L = 16
BPW = B // NW
CH = 64
NR = BPW // CH
GPR = CH // L
NIC = BPW // 128

TBLK = 16384


def _wbuild_body(ut_ref, vt_ref, e1_ref, e2_ref, o_ref):
    dn = (((0,), (0,)), ((), ()))
    o_ref[...] = (
        lax.dot_general(ut_ref[...], e1_ref[...], dn,
                        preferred_element_type=jnp.float32)
        + lax.dot_general(vt_ref[...], e2_ref[...], dn,
                          preferred_element_type=jnp.float32))


RPR = (2 + NEG) * CH


def _sc_body(m3, W_hbm, out_hbm, midx, rows_v, sbuf, sem0, sem1):
    wid = lax.axis_index("s") * NC + lax.axis_index("c")

    pltpu.sync_copy(m3.at[wid], midx)

    lane = lax.iota(jnp.int32, L)
    sems = (sem0, sem1)

    def fire(r):
        b = r % 2
        s = sems[b]
        cps = []
        for (o, n) in ((0, 128), (128, 128), (256, 128), (384, 64)):
            cps.append(pltpu.async_copy(
                W_hbm.at[midx.at[r, pl.ds(o, n)]],
                rows_v.at[b, pl.ds(o, n)], s))
        return cps

    pend = fire(0)
    for r in range(NR):
        nxt = fire(r + 1) if r + 1 < NR else []
        for c in pend:
            c.wait()
        pend = nxt
        b = r % 2
        bvec = jnp.full((L,), b, jnp.int32)

        def group_body(go, _):
            rows = go * L + lane

            def d_body(d, carry):
                acc_p, acc_n = carry
                du = jnp.full((L,), d, jnp.int32)
                dv = du + D
                ucol = plsc.load_gather(rows_v, [bvec, rows, du])
                vcol = plsc.load_gather(rows_v, [bvec, rows + CH, dv])
                ncol = plsc.load_gather(rows_v, [bvec, rows + 2 * CH, dv])
                for k in range(1, NEG):
                    ncol = ncol + plsc.load_gather(
                        rows_v, [bvec, rows + (2 + k) * CH, dv])
                return acc_p + ucol * vcol, acc_n + ucol * ncol

            z = jnp.zeros((L,), jnp.float32)
            acc_p, acc_n = lax.fori_loop(0, D, d_body, (z, z), unroll=8)
            off = (r & 1) * CH + go * L
            sbuf[r >> 1, pl.ds(off, L)] = acc_p
            sbuf[NIC + (r >> 1), pl.ds(off, L)] = acc_n
            return 0

        lax.fori_loop(0, GPR, group_body, 0)

    pltpu.sync_copy(sbuf, out_hbm.at[wid])


def _loss_body(x_ref, o_ref):
    s = x_ref[:, 0:NIC, :]
    n = -x_ref[:, NIC:2 * NIC, :]

    def ls(x):
        return jnp.minimum(x, 0.0) - jnp.log1p(jnp.exp(-jnp.abs(x)))

    o_ref[...] = (-(jnp.sum(ls(s) + ls(n))) / B).reshape(1, 1)


def kernel(u, v, neg_v, U, V):
    eye = jnp.eye(D, dtype=jnp.float32)
    zer = jnp.zeros((D, D), jnp.float32)
    e1 = jnp.concatenate([eye, zer], axis=1)
    e2 = jnp.concatenate([zer, eye], axis=1)
    W = pl.pallas_call(
        _wbuild_body,
        grid=(pl.cdiv(VOCAB, TBLK),),
        in_specs=[
            pl.BlockSpec((D, TBLK), lambda j: (0, j)),
            pl.BlockSpec((D, TBLK), lambda j: (0, j)),
            pl.BlockSpec((D, 2 * D), lambda j: (0, 0)),
            pl.BlockSpec((D, 2 * D), lambda j: (0, 0)),
        ],
        out_specs=pl.BlockSpec((TBLK, 2 * D), lambda j: (j, 0)),
        out_shape=jax.ShapeDtypeStruct((VOCAB, 2 * D), jnp.float32),
    )(U.T, V.T, e1, e2)

    m3 = jnp.concatenate(
        [u.astype(jnp.int32).reshape(NW, NR, CH),
         v.astype(jnp.int32).reshape(NW, NR, CH),
         neg_v.astype(jnp.int32).T.reshape(NEG, NW, NR, CH)
         .transpose(1, 2, 0, 3).reshape(NW, NR, NEG * CH)],
        axis=2)

    mesh = plsc.VectorSubcoreMesh(core_axis_name="c", subcore_axis_name="s")
    packed = pl.kernel(
        _sc_body,
        out_type=jax.ShapeDtypeStruct((NW, 2 * NIC, 128), jnp.float32),
        mesh=mesh,
        compiler_params=pltpu.CompilerParams(needs_layout_passes=False),
        scratch_types=[
            pltpu.VMEM((NR, RPR), jnp.int32),
            pltpu.VMEM((2, RPR, 2 * D), jnp.float32),
            pltpu.VMEM((2 * NIC, 128), jnp.float32),
            pltpu.SemaphoreType.DMA,
            pltpu.SemaphoreType.DMA,
        ],
    )(m3, W)

    loss = pl.pallas_call(
        _loss_body,
        out_shape=jax.ShapeDtypeStruct((1, 1), jnp.float32),
    )(packed)
    return loss[0, 0]
